# baseline (speedup 1.0000x reference)
import functools

import jax
import jax.numpy as jnp
from jax import lax
from jax.experimental import pallas as pl
from jax.experimental.pallas import tpu as pltpu
from jax.experimental.pallas import tpu_sc as plsc

N_NODES = 50000
DIM = 64
HALF = 25000
NC, NS, LANES = 2, 16, 16
CHUNK = 128
E_EDGES = 800000
CPT = 400
E_PAD = NS * CPT * CHUNK
ACC_ROWS = 25088
DUMMY = 25024
NPAD = 50176
BLK = 1024
GRID = NPAD // BLK
IDXB = 8
NBI = CPT // IDXB
KBD = 10
NBD = CPT // KBD
WB = 1560

_sc_mesh = plsc.VectorSubcoreMesh(core_axis_name="c", subcore_axis_name="s")
_sc_params = pltpu.CompilerParams(use_tc_tiling_on_sc=False)


@functools.partial(
    pl.kernel,
    out_type=jax.ShapeDtypeStruct((NPAD, LANES), jnp.float32),
    mesh=_sc_mesh,
    scratch_types=[
        pltpu.VMEM((2, KBD * CHUNK), jnp.int32),
        pltpu.VMEM((2, KBD, CHUNK), jnp.int32),
        pltpu.VMEM((CHUNK, LANES), jnp.float32),
        pltpu.VMEM((WB, LANES), jnp.float32),
        pltpu.SemaphoreType.DMA,
        pltpu.SemaphoreType.DMA,
        pltpu.VMEM_SHARED((ACC_ROWS, LANES), jnp.float32),
    ],
    compiler_params=_sc_params,
)
def _deg_kernel(dst_hbm, cnt_hbm, dst_v, ldst_v, ones_v, zbuf_v,
                sem_i, sem_s, acc_s):
    c = lax.axis_index("c")
    s = lax.axis_index("s")
    base = c * HALF
    tbase = s * CPT * CHUNK
    r0 = s * WB

    def fill_ones(i, carry):
        ones_v[i, :] = jnp.ones((LANES,), jnp.float32)
        return carry

    lax.fori_loop(0, CHUNK, fill_ones, 0)

    def fill_zero(i, carry):
        zbuf_v[i, :] = jnp.zeros((LANES,), jnp.float32)
        return carry

    lax.fori_loop(0, WB, fill_zero, 0)
    pltpu.sync_copy(zbuf_v, acc_s.at[pl.ds(r0, WB)])

    @pl.when(s == NS - 1)
    def _():
        pltpu.sync_copy(zbuf_v.at[pl.ds(0, 40)], acc_s.at[pl.ds(NS * WB, 40)])

    plsc.subcore_barrier()

    def issue_idx(p, b):
        off = tbase + b * KBD * CHUNK
        pltpu.async_copy(dst_hbm.at[pl.ds(off, KBD * CHUNK)], dst_v.at[p], sem_i)

    def wait_idx(p):
        pltpu.make_async_copy(dst_hbm.at[pl.ds(0, KBD * CHUNK)], dst_v.at[p],
                              sem_i).wait()

    def remap(p):
        for k in range(KBD):
            for g in range(CHUNK // LANES):
                d = dst_v[p, pl.ds(k * CHUNK + g * LANES, LANES)]
                ok = (d >= base) & (d < base + HALF)
                ldst_v[p, k, pl.ds(g * LANES, LANES)] = jnp.where(ok, d - base, DUMMY)

    def fire(p):
        for k in range(KBD):
            pltpu.async_copy(ones_v, acc_s.at[ldst_v.at[p, k]], sem_s, add=True)

    def drain(p):
        for k in range(KBD):
            pltpu.make_async_copy(ones_v, acc_s.at[ldst_v.at[p, k]], sem_s).wait()

    issue_idx(0, 0)
    issue_idx(1, 1)

    def body(i, carry):
        wait_idx(0)
        remap(0)

        @pl.when(i < NBD // 2 - 1)
        def _():
            issue_idx(0, 2 * i + 2)

        @pl.when(i > 0)
        def _():
            drain(1)

        fire(0)
        wait_idx(1)
        remap(1)

        @pl.when(i < NBD // 2 - 1)
        def _():
            issue_idx(1, 2 * i + 3)

        drain(0)
        fire(1)
        return carry

    lax.fori_loop(0, NBD // 2, body, 0)
    drain(1)
    plsc.subcore_barrier()
    pltpu.sync_copy(acc_s.at[pl.ds(r0, WB)], cnt_hbm.at[pl.ds(base + r0, WB)])

    @pl.when(s == NS - 1)
    def _():
        pltpu.sync_copy(acc_s.at[pl.ds(NS * WB, 40)],
                        cnt_hbm.at[pl.ds(base + NS * WB, 40)])


@functools.partial(
    pl.kernel,
    out_type=jax.ShapeDtypeStruct((NPAD, DIM), jnp.float32),
    mesh=_sc_mesh,
    scratch_types=[
        pltpu.VMEM((2, IDXB * CHUNK), jnp.int32),
        pltpu.VMEM((2, IDXB * CHUNK), jnp.int32),
        pltpu.VMEM((2, IDXB, CHUNK), jnp.int32),
        pltpu.VMEM((2, CHUNK, DIM), jnp.float32),
        pltpu.SemaphoreType.DMA,
        pltpu.SemaphoreType.DMA,
        pltpu.SemaphoreType.DMA,
        pltpu.SemaphoreType.DMA,
        pltpu.VMEM_SHARED((ACC_ROWS, DIM), jnp.float32),
    ],
    compiler_params=_sc_params,
)
def _scatter_kernel(y_hbm, src_hbm, dst_hbm, out_hbm,
                    src_v, dst_v, ldst_v, rows_v, sem_i, sem_g, sem_s0, sem_s1,
                    acc_s):
    sem_s = (sem_s0, sem_s1)
    c = lax.axis_index("c")
    s = lax.axis_index("s")
    base = c * HALF
    tbase = s * CPT * CHUNK
    r0 = s * WB
    pltpu.sync_copy(y_hbm.at[pl.ds(base + r0, WB)], acc_s.at[pl.ds(r0, WB)])

    @pl.when(s == NS - 1)
    def _():
        pltpu.sync_copy(y_hbm.at[pl.ds(base + NS * WB, 40)],
                        acc_s.at[pl.ds(NS * WB, 40)])

    plsc.subcore_barrier()

    def issue_idx(p, b):
        off = tbase + b * IDXB * CHUNK
        pltpu.async_copy(src_hbm.at[pl.ds(off, IDXB * CHUNK)], src_v.at[p], sem_i)
        pltpu.async_copy(dst_hbm.at[pl.ds(off, IDXB * CHUNK)], dst_v.at[p], sem_i)

    def wait_idx(p):
        pltpu.make_async_copy(src_hbm.at[pl.ds(0, IDXB * CHUNK)], src_v.at[p],
                              sem_i).wait()
        pltpu.make_async_copy(dst_hbm.at[pl.ds(0, IDXB * CHUNK)], dst_v.at[p],
                              sem_i).wait()

    def remap(p):
        for k in range(IDXB):
            for g in range(CHUNK // LANES):
                d = dst_v[p, pl.ds(k * CHUNK + g * LANES, LANES)]
                ok = (d >= base) & (d < base + HALF)
                ldst_v[p, k, pl.ds(g * LANES, LANES)] = jnp.where(ok, d - base, DUMMY)

    def fire_gather(p, r, k):
        pltpu.async_copy(y_hbm.at[src_v.at[p, pl.ds(k * CHUNK, CHUNK)]],
                         rows_v.at[r], sem_g)

    def drain_gather(p, r, k):
        pltpu.make_async_copy(y_hbm.at[src_v.at[p, pl.ds(k * CHUNK, CHUNK)]],
                              rows_v.at[r], sem_g).wait()

    def fire_scatter(p, r, k):
        pltpu.async_copy(rows_v.at[r], acc_s.at[ldst_v.at[p, k]], sem_s[r],
                         add=True)

    def drain_scatter(p, r, k):
        pltpu.make_async_copy(rows_v.at[r], acc_s.at[ldst_v.at[p, k]],
                              sem_s[r]).wait()

    issue_idx(0, 0)
    issue_idx(1, 1)

    def body(i, carry):
        for p in (0, 1):
            b = 2 * i + p
            wait_idx(p)
            remap(p)

            for k in range(IDXB):
                r = k % 2
                if p == 0 and k < 2:
                    @pl.when(i > 0)
                    def _():
                        drain_scatter(p, r, k)
                else:
                    drain_scatter(p, r, k)
                fire_gather(p, r, k)
                drain_gather(p, r, k)
                fire_scatter(p, r, k)

            @pl.when(i < NBI // 2 - 1)
            def _():
                issue_idx(p, b + 2)
        return carry

    lax.fori_loop(0, NBI // 2, body, 0)
    drain_scatter(1, 0, IDXB - 2)
    drain_scatter(1, 1, IDXB - 1)
    plsc.subcore_barrier()
    pltpu.sync_copy(acc_s.at[pl.ds(r0, WB)], out_hbm.at[pl.ds(base + r0, WB)])

    @pl.when(s == NS - 1)
    def _():
        pltpu.sync_copy(acc_s.at[pl.ds(NS * WB, 40)],
                        out_hbm.at[pl.ds(base + NS * WB, 40)])


def _tcA_body(x_ref, cnt_ref, w_ref, y_ref, dis_ref):
    dis = lax.rsqrt(cnt_ref[...][:, 0:1] + 1.0)
    xw = jnp.dot(x_ref[...], w_ref[...], preferred_element_type=jnp.float32)
    y_ref[...] = xw * dis
    dis_ref[...] = dis


def _tcB_body(acc_ref, dis_ref, b_ref, w_ref, out_ref, y_ref):
    dis = dis_ref[...]
    out = acc_ref[...] * dis + b_ref[0:1, :]
    out_ref[...] = out
    h = jnp.maximum(out, 0.0)
    y_ref[...] = jnp.dot(h, w_ref[...], preferred_element_type=jnp.float32) * dis


def _tcC_body(acc_ref, dis_ref, b_ref, out_ref):
    out_ref[...] = acc_ref[...] * dis_ref[...] + b_ref[0:1, :]


def _tcA(x_pad, cnt, W1):
    return pl.pallas_call(
        _tcA_body,
        grid=(GRID,),
        in_specs=[pl.BlockSpec((BLK, DIM), lambda i: (i, 0)),
                  pl.BlockSpec((BLK, LANES), lambda i: (i, 0)),
                  pl.BlockSpec((DIM, DIM), lambda i: (0, 0))],
        out_specs=[pl.BlockSpec((BLK, DIM), lambda i: (i, 0)),
                   pl.BlockSpec((BLK, 1), lambda i: (i, 0))],
        out_shape=[jax.ShapeDtypeStruct((NPAD, DIM), jnp.float32),
                   jax.ShapeDtypeStruct((NPAD, 1), jnp.float32)],
    )(x_pad, cnt, W1)


def _tcB(acc, dis, b1b, W2):
    return pl.pallas_call(
        _tcB_body,
        grid=(GRID,),
        in_specs=[pl.BlockSpec((BLK, DIM), lambda i: (i, 0)),
                  pl.BlockSpec((BLK, 1), lambda i: (i, 0)),
                  pl.BlockSpec((8, DIM), lambda i: (0, 0)),
                  pl.BlockSpec((DIM, DIM), lambda i: (0, 0))],
        out_specs=[pl.BlockSpec((BLK, DIM), lambda i: (i, 0)),
                   pl.BlockSpec((BLK, DIM), lambda i: (i, 0))],
        out_shape=[jax.ShapeDtypeStruct((NPAD, DIM), jnp.float32),
                   jax.ShapeDtypeStruct((NPAD, DIM), jnp.float32)],
    )(acc, dis, b1b, W2)


def _tcC(acc, dis, b2b):
    return pl.pallas_call(
        _tcC_body,
        grid=(GRID,),
        in_specs=[pl.BlockSpec((BLK, DIM), lambda i: (i, 0)),
                  pl.BlockSpec((BLK, 1), lambda i: (i, 0)),
                  pl.BlockSpec((8, DIM), lambda i: (0, 0))],
        out_specs=pl.BlockSpec((BLK, DIM), lambda i: (i, 0)),
        out_shape=jax.ShapeDtypeStruct((NPAD, DIM), jnp.float32),
    )(acc, dis, b2b)


def kernel(edge_index, edge_weight, emb_users, emb_items, W1, b1, W2, b2):
    del edge_weight
    src = edge_index[0].astype(jnp.int32)
    dst = edge_index[1].astype(jnp.int32)
    pad_e = E_PAD - E_EDGES
    src_p = jnp.concatenate([src, jnp.full((pad_e,), N_NODES, jnp.int32)])
    dst_p = jnp.concatenate([dst, jnp.full((pad_e,), 2 ** 28, jnp.int32)])

    x = jnp.concatenate([emb_users, emb_items], axis=0)
    x_pad = jnp.concatenate([x, jnp.zeros((NPAD - N_NODES, DIM), jnp.float32)])

    cnt = _deg_kernel(dst_p)
    y1, dis = _tcA(x_pad, cnt, W1)
    acc1 = _scatter_kernel(y1, src_p, dst_p)
    b1b = jnp.broadcast_to(b1.reshape(1, DIM), (8, DIM))
    out1, y2 = _tcB(acc1, dis, b1b, W2)
    acc2 = _scatter_kernel(y2, src_p, dst_p)
    b2b = jnp.broadcast_to(b2.reshape(1, DIM), (8, DIM))
    out2 = _tcC(acc2, dis, b2b)

    return (x, out1[:N_NODES], out2[:N_NODES])


# speedup vs baseline: 1.2091x; 1.2091x over previous
import functools

import jax
import jax.numpy as jnp
from jax import lax
from jax.experimental import pallas as pl
from jax.experimental.pallas import tpu as pltpu
from jax.experimental.pallas import tpu_sc as plsc

N_NODES = 50000
DIM = 64
HALF = 25000
NC, NS, LANES = 2, 16, 16
CHUNK = 128
E_EDGES = 800000
CPT = 396
E_PAD = NS * CPT * CHUNK
ACC_ROWS = 25088
DUMMY = 25024
NPAD = 50176
BLK = 1024
GRID = NPAD // BLK
IDXB = 4
NBI = CPT // IDXB
KBD = 11
NBD = CPT // KBD
WB = 1560

_sc_mesh = plsc.VectorSubcoreMesh(core_axis_name="c", subcore_axis_name="s")
_sc_params = pltpu.CompilerParams(use_tc_tiling_on_sc=False)


@functools.partial(
    pl.kernel,
    out_type=jax.ShapeDtypeStruct((NPAD, LANES), jnp.float32),
    mesh=_sc_mesh,
    scratch_types=[
        pltpu.VMEM((2, KBD * CHUNK), jnp.int32),
        pltpu.VMEM((2, KBD, CHUNK), jnp.int32),
        pltpu.VMEM((CHUNK, LANES), jnp.float32),
        pltpu.VMEM((WB, LANES), jnp.float32),
        pltpu.SemaphoreType.DMA,
        pltpu.SemaphoreType.DMA,
        pltpu.VMEM_SHARED((ACC_ROWS, LANES), jnp.float32),
    ],
    compiler_params=_sc_params,
)
def _deg_kernel(dst_hbm, cnt_hbm, dst_v, ldst_v, ones_v, zbuf_v,
                sem_i, sem_s, acc_s):
    c = lax.axis_index("c")
    s = lax.axis_index("s")
    base = c * HALF
    tbase = s * CPT * CHUNK
    r0 = s * WB

    def fill_ones(i, carry):
        ones_v[i, :] = jnp.ones((LANES,), jnp.float32)
        return carry

    lax.fori_loop(0, CHUNK, fill_ones, 0)

    def fill_zero(i, carry):
        zbuf_v[i, :] = jnp.zeros((LANES,), jnp.float32)
        return carry

    lax.fori_loop(0, WB, fill_zero, 0)
    pltpu.sync_copy(zbuf_v, acc_s.at[pl.ds(r0, WB)])

    @pl.when(s == NS - 1)
    def _():
        pltpu.sync_copy(zbuf_v.at[pl.ds(0, 40)], acc_s.at[pl.ds(NS * WB, 40)])

    plsc.subcore_barrier()

    def issue_idx(p, b):
        off = tbase + b * KBD * CHUNK
        pltpu.async_copy(dst_hbm.at[pl.ds(off, KBD * CHUNK)], dst_v.at[p], sem_i)

    def wait_idx(p):
        pltpu.make_async_copy(dst_hbm.at[pl.ds(0, KBD * CHUNK)], dst_v.at[p],
                              sem_i).wait()

    def remap(p):
        for k in range(KBD):
            for g in range(CHUNK // LANES):
                d = dst_v[p, pl.ds(k * CHUNK + g * LANES, LANES)]
                ok = (d >= base) & (d < base + HALF)
                ldst_v[p, k, pl.ds(g * LANES, LANES)] = jnp.where(ok, d - base, DUMMY)

    def fire(p):
        for k in range(KBD):
            pltpu.async_copy(ones_v, acc_s.at[ldst_v.at[p, k]], sem_s, add=True)

    def drain(p):
        for k in range(KBD):
            pltpu.make_async_copy(ones_v, acc_s.at[ldst_v.at[p, k]], sem_s).wait()

    issue_idx(0, 0)
    issue_idx(1, 1)

    def body(i, carry):
        wait_idx(0)
        remap(0)

        @pl.when(i < NBD // 2 - 1)
        def _():
            issue_idx(0, 2 * i + 2)

        @pl.when(i > 0)
        def _():
            drain(1)

        fire(0)
        wait_idx(1)
        remap(1)

        @pl.when(i < NBD // 2 - 1)
        def _():
            issue_idx(1, 2 * i + 3)

        drain(0)
        fire(1)
        return carry

    lax.fori_loop(0, NBD // 2, body, 0)
    drain(1)
    plsc.subcore_barrier()
    pltpu.sync_copy(acc_s.at[pl.ds(r0, WB)], cnt_hbm.at[pl.ds(base + r0, WB)])

    @pl.when(s == NS - 1)
    def _():
        pltpu.sync_copy(acc_s.at[pl.ds(NS * WB, 40)],
                        cnt_hbm.at[pl.ds(base + NS * WB, 40)])


@functools.partial(
    pl.kernel,
    out_type=jax.ShapeDtypeStruct((NPAD, DIM), jnp.float32),
    mesh=_sc_mesh,
    scratch_types=[
        pltpu.VMEM((3, IDXB * CHUNK), jnp.int32),
        pltpu.VMEM((3, IDXB * CHUNK), jnp.int32),
        pltpu.VMEM((3, IDXB, CHUNK), jnp.int32),
        pltpu.VMEM((3, CHUNK, DIM), jnp.float32),
        pltpu.SemaphoreType.DMA,
        pltpu.SemaphoreType.DMA,
        pltpu.SemaphoreType.DMA,
        pltpu.SemaphoreType.DMA,
        pltpu.SemaphoreType.DMA,
        pltpu.SemaphoreType.DMA,
        pltpu.SemaphoreType.DMA,
        pltpu.VMEM_SHARED((ACC_ROWS, DIM), jnp.float32),
    ],
    compiler_params=_sc_params,
)
def _scatter_kernel(y_hbm, src_hbm, dst_hbm, out_hbm,
                    src_v, dst_v, ldst_v, rows_v, sem_i,
                    sem_g0, sem_g1, sem_g2, sem_s0, sem_s1, sem_s2, acc_s):
    sem_g = (sem_g0, sem_g1, sem_g2)
    sem_s = (sem_s0, sem_s1, sem_s2)
    c = lax.axis_index("c")
    s = lax.axis_index("s")
    base = c * HALF
    tbase = s * CPT * CHUNK
    r0 = s * WB
    pltpu.sync_copy(y_hbm.at[pl.ds(base + r0, WB)], acc_s.at[pl.ds(r0, WB)])

    @pl.when(s == NS - 1)
    def _():
        pltpu.sync_copy(y_hbm.at[pl.ds(base + NS * WB, 40)],
                        acc_s.at[pl.ds(NS * WB, 40)])

    plsc.subcore_barrier()

    def issue_idx(p, b):
        off = tbase + b * IDXB * CHUNK
        pltpu.async_copy(src_hbm.at[pl.ds(off, IDXB * CHUNK)], src_v.at[p], sem_i)
        pltpu.async_copy(dst_hbm.at[pl.ds(off, IDXB * CHUNK)], dst_v.at[p], sem_i)

    def wait_idx(p):
        pltpu.make_async_copy(src_hbm.at[pl.ds(0, IDXB * CHUNK)], src_v.at[p],
                              sem_i).wait()
        pltpu.make_async_copy(dst_hbm.at[pl.ds(0, IDXB * CHUNK)], dst_v.at[p],
                              sem_i).wait()

    def remap(p):
        for k in range(IDXB):
            for g in range(CHUNK // LANES):
                d = dst_v[p, pl.ds(k * CHUNK + g * LANES, LANES)]
                ok = (d >= base) & (d < base + HALF)
                ldst_v[p, k, pl.ds(g * LANES, LANES)] = jnp.where(ok, d - base, DUMMY)

    def fire_gather(p, r, k):
        pltpu.async_copy(y_hbm.at[src_v.at[p, pl.ds(k * CHUNK, CHUNK)]],
                         rows_v.at[r], sem_g[r])

    def drain_gather(p, r, k):
        pltpu.make_async_copy(y_hbm.at[src_v.at[p, pl.ds(k * CHUNK, CHUNK)]],
                              rows_v.at[r], sem_g[r]).wait()

    def fire_scatter(p, r, k):
        pltpu.async_copy(rows_v.at[r], acc_s.at[ldst_v.at[p, k]], sem_s[r],
                         add=True)

    def drain_scatter(p, r, k):
        pltpu.make_async_copy(rows_v.at[r], acc_s.at[ldst_v.at[p, k]],
                              sem_s[r]).wait()

    issue_idx(0, 0)
    issue_idx(1, 1)
    issue_idx(2, 2)

    def body(i, carry):
        for q in (0, 1, 2):
            b = 3 * i + q
            wait_idx(q)
            remap(q)

            for k in range(IDXB):
                r = (4 * q + k) % 3
                q1, k1 = (q, k - 1) if k > 0 else ((q + 2) % 3, IDXB - 1)
                r1 = (4 * q1 + k1) % 3

                if q == 0 and k < 3:
                    @pl.when(i > 0)
                    def _():
                        drain_scatter(r, r, 0)
                else:
                    drain_scatter(r, r, 0)
                fire_gather(q, r, k)
                if q == 0 and k == 0:
                    @pl.when(i > 0)
                    def _():
                        drain_gather(q1, r1, k1)
                        fire_scatter(q1, r1, k1)
                else:
                    drain_gather(q1, r1, k1)
                    fire_scatter(q1, r1, k1)

            @pl.when(jnp.logical_and(b >= 1, b + 2 < NBI))
            def _():
                issue_idx((q + 2) % 3, b + 2)
        return carry

    lax.fori_loop(0, NBI // 3, body, 0)
    qL, kL = 2, IDXB - 1
    rL = (4 * qL + kL) % 3
    drain_gather(qL, rL, kL)
    fire_scatter(qL, rL, kL)
    for r in (0, 1, 2):
        drain_scatter(r, r, 0)
    plsc.subcore_barrier()
    pltpu.sync_copy(acc_s.at[pl.ds(r0, WB)], out_hbm.at[pl.ds(base + r0, WB)])

    @pl.when(s == NS - 1)
    def _():
        pltpu.sync_copy(acc_s.at[pl.ds(NS * WB, 40)],
                        out_hbm.at[pl.ds(base + NS * WB, 40)])


def _tcA_body(x_ref, cnt_ref, w_ref, y_ref, dis_ref):
    dis = lax.rsqrt(cnt_ref[...][:, 0:1] + 1.0)
    xw = jnp.dot(x_ref[...], w_ref[...], preferred_element_type=jnp.float32)
    y_ref[...] = xw * dis
    dis_ref[...] = dis


def _tcB_body(acc_ref, dis_ref, b_ref, w_ref, out_ref, y_ref):
    dis = dis_ref[...]
    out = acc_ref[...] * dis + b_ref[0:1, :]
    out_ref[...] = out
    h = jnp.maximum(out, 0.0)
    y_ref[...] = jnp.dot(h, w_ref[...], preferred_element_type=jnp.float32) * dis


def _tcC_body(acc_ref, dis_ref, b_ref, out_ref):
    out_ref[...] = acc_ref[...] * dis_ref[...] + b_ref[0:1, :]


def _tcA(x_pad, cnt, W1):
    return pl.pallas_call(
        _tcA_body,
        grid=(GRID,),
        in_specs=[pl.BlockSpec((BLK, DIM), lambda i: (i, 0)),
                  pl.BlockSpec((BLK, LANES), lambda i: (i, 0)),
                  pl.BlockSpec((DIM, DIM), lambda i: (0, 0))],
        out_specs=[pl.BlockSpec((BLK, DIM), lambda i: (i, 0)),
                   pl.BlockSpec((BLK, 1), lambda i: (i, 0))],
        out_shape=[jax.ShapeDtypeStruct((NPAD, DIM), jnp.float32),
                   jax.ShapeDtypeStruct((NPAD, 1), jnp.float32)],
    )(x_pad, cnt, W1)


def _tcB(acc, dis, b1b, W2):
    return pl.pallas_call(
        _tcB_body,
        grid=(GRID,),
        in_specs=[pl.BlockSpec((BLK, DIM), lambda i: (i, 0)),
                  pl.BlockSpec((BLK, 1), lambda i: (i, 0)),
                  pl.BlockSpec((8, DIM), lambda i: (0, 0)),
                  pl.BlockSpec((DIM, DIM), lambda i: (0, 0))],
        out_specs=[pl.BlockSpec((BLK, DIM), lambda i: (i, 0)),
                   pl.BlockSpec((BLK, DIM), lambda i: (i, 0))],
        out_shape=[jax.ShapeDtypeStruct((NPAD, DIM), jnp.float32),
                   jax.ShapeDtypeStruct((NPAD, DIM), jnp.float32)],
    )(acc, dis, b1b, W2)


def _tcC(acc, dis, b2b):
    return pl.pallas_call(
        _tcC_body,
        grid=(GRID,),
        in_specs=[pl.BlockSpec((BLK, DIM), lambda i: (i, 0)),
                  pl.BlockSpec((BLK, 1), lambda i: (i, 0)),
                  pl.BlockSpec((8, DIM), lambda i: (0, 0))],
        out_specs=pl.BlockSpec((BLK, DIM), lambda i: (i, 0)),
        out_shape=jax.ShapeDtypeStruct((NPAD, DIM), jnp.float32),
    )(acc, dis, b2b)


def kernel(edge_index, edge_weight, emb_users, emb_items, W1, b1, W2, b2):
    del edge_weight
    src = edge_index[0].astype(jnp.int32)
    dst = edge_index[1].astype(jnp.int32)
    pad_e = E_PAD - E_EDGES
    src_p = jnp.concatenate([src, jnp.full((pad_e,), N_NODES, jnp.int32)])
    dst_p = jnp.concatenate([dst, jnp.full((pad_e,), 2 ** 28, jnp.int32)])

    x = jnp.concatenate([emb_users, emb_items], axis=0)
    x_pad = jnp.concatenate([x, jnp.zeros((NPAD - N_NODES, DIM), jnp.float32)])

    cnt = _deg_kernel(dst_p)
    y1, dis = _tcA(x_pad, cnt, W1)
    acc1 = _scatter_kernel(y1, src_p, dst_p)
    b1b = jnp.broadcast_to(b1.reshape(1, DIM), (8, DIM))
    out1, y2 = _tcB(acc1, dis, b1b, W2)
    acc2 = _scatter_kernel(y2, src_p, dst_p)
    b2b = jnp.broadcast_to(b2.reshape(1, DIM), (8, DIM))
    out2 = _tcC(acc2, dis, b2b)

    return (x, out1[:N_NODES], out2[:N_NODES])


# speedup vs baseline: 2.5263x; 2.0894x over previous
import functools

import jax
import jax.numpy as jnp
from jax import lax
from jax.experimental import pallas as pl
from jax.experimental.pallas import tpu as pltpu
from jax.experimental.pallas import tpu_sc as plsc

N_NODES = 50000
DIM = 64
HALF = 25000
NC, NS, LANES = 2, 16, 16
NW = NC * NS
CHUNK = 128
E_EDGES = 800000
CPT = 408
E_PAD = NS * CPT * CHUNK
KP = 12
NBP = CPT // KP
SLAB = 2048
NCH = SLAB // CHUNK
SCAP = 2944
PADN = 768
CAP_T = CPT * CHUNK + PADN + 2 * SLAB
NTCH = 6
ACC_ROWS = 25088
DUMMY = 25024
NPAD = 50176
BLK = 1024
GRID = NPAD // BLK
WB = 1560

_sc_mesh = plsc.VectorSubcoreMesh(core_axis_name="c", subcore_axis_name="s")
_sc_params = pltpu.CompilerParams(use_tc_tiling_on_sc=False,
                                  needs_layout_passes=False)


@functools.partial(
    pl.kernel,
    out_type=[jax.ShapeDtypeStruct((NPAD, LANES), jnp.float32),
              jax.ShapeDtypeStruct((NW, CAP_T), jnp.int32),
              jax.ShapeDtypeStruct((NW, CAP_T), jnp.int32),
              jax.ShapeDtypeStruct((NW, 16), jnp.int32)],
    mesh=_sc_mesh,
    scratch_types=[
        pltpu.VMEM((2, KP * CHUNK), jnp.int32),
        pltpu.VMEM((2, KP * CHUNK), jnp.int32),
        pltpu.VMEM((SCAP,), jnp.int32),
        pltpu.VMEM((SCAP,), jnp.int32),
        pltpu.VMEM((NCH, CHUNK), jnp.int32),
        pltpu.VMEM((CHUNK, LANES), jnp.float32),
        pltpu.VMEM((WB, LANES), jnp.float32),
        pltpu.VMEM((16,), jnp.int32),
        pltpu.SemaphoreType.DMA,
        pltpu.SemaphoreType.DMA,
        pltpu.VMEM_SHARED((ACC_ROWS, LANES), jnp.float32),
    ],
    compiler_params=_sc_params,
)
def _part_kernel(src_hbm, dst_hbm, cnt_hbm, psrc_hbm, pldst_hbm, pcnt_hbm,
                 srci_v, dsti_v, ssrc, sldst, lbuf, ones_v, zbuf_v, ncbuf,
                 sem_i, sem_s, acc_s):
    c = lax.axis_index("c")
    s = lax.axis_index("s")
    w = c * NS + s
    base = c * HALF
    tbase = s * CPT * CHUNK
    r0 = s * WB

    def fill_ones(i, carry):
        ones_v[i, :] = jnp.ones((LANES,), jnp.float32)
        return carry

    lax.fori_loop(0, CHUNK, fill_ones, 0)

    def fill_zero(i, carry):
        zbuf_v[i, :] = jnp.zeros((LANES,), jnp.float32)
        return carry

    lax.fori_loop(0, WB, fill_zero, 0)
    pltpu.sync_copy(zbuf_v, acc_s.at[pl.ds(r0, WB)])

    @pl.when(s == NS - 1)
    def _():
        pltpu.sync_copy(zbuf_v.at[pl.ds(0, 40)], acc_s.at[pl.ds(NS * WB, 40)])

    plsc.subcore_barrier()

    def issue_idx(p, b):
        off_e = tbase + b * KP * CHUNK
        pltpu.async_copy(src_hbm.at[pl.ds(off_e, KP * CHUNK)], srci_v.at[p], sem_i)
        pltpu.async_copy(dst_hbm.at[pl.ds(off_e, KP * CHUNK)], dsti_v.at[p], sem_i)

    def wait_idx(p):
        pltpu.make_async_copy(src_hbm.at[pl.ds(0, KP * CHUNK)], srci_v.at[p],
                              sem_i).wait()
        pltpu.make_async_copy(dst_hbm.at[pl.ds(0, KP * CHUNK)], dsti_v.at[p],
                              sem_i).wait()

    def fire_ones(ch):
        pltpu.async_copy(ones_v, acc_s.at[lbuf.at[ch]], sem_s, add=True)

    def drain_ones(ch):
        pltpu.make_async_copy(ones_v, acc_s.at[lbuf.at[ch]], sem_s).wait()

    def flush(off, wp, tail_groups):
        @pl.when(wp > 0)
        def _():
            for ch in range(NCH):
                drain_ones(ch)
        for ch in range(NCH):
            for g in range(CHUNK // LANES):
                lbuf[ch, pl.ds(g * LANES, LANES)] = (
                    sldst[pl.ds(ch * CHUNK + g * LANES, LANES)])
        wpa = pl.multiple_of(wp, SLAB)
        pltpu.sync_copy(ssrc.at[pl.ds(0, SLAB)], psrc_hbm.at[w, pl.ds(wpa, SLAB)])
        pltpu.sync_copy(sldst.at[pl.ds(0, SLAB)], pldst_hbm.at[w, pl.ds(wpa, SLAB)])
        for ch in range(NCH):
            fire_ones(ch)
        for g in range(tail_groups):
            sv = ssrc[pl.ds(SLAB + g * LANES, LANES)]
            lv = sldst[pl.ds(SLAB + g * LANES, LANES)]
            ssrc[pl.ds(g * LANES, LANES)] = sv
            sldst[pl.ds(g * LANES, LANES)] = lv

    issue_idx(0, 0)
    issue_idx(1, 1)

    def scan_body(i, carry):
        off, wp = carry
        for p in (0, 1):
            b = 2 * i + p
            wait_idx(p)
            for k in range(KP):
                for g in range(CHUNK // LANES):
                    d = dsti_v[p, pl.ds(k * CHUNK + g * LANES, LANES)]
                    sr = srci_v[p, pl.ds(k * CHUNK + g * LANES, LANES)]
                    ok = jnp.logical_and(d >= base, d < base + HALF)
                    plsc.store_compressed(sldst.at[pl.ds(off, LANES)],
                                          d - base, mask=ok)
                    plsc.store_compressed(ssrc.at[pl.ds(off, LANES)], sr,
                                          mask=ok)
                    off = off + jnp.max(plsc.all_reduce_population_count(ok))
                if k % 4 == 3:
                    do_flush = off >= SLAB

                    @pl.when(do_flush)
                    def _():
                        flush(off, wp, 32)

                    off = jnp.where(do_flush, off - SLAB, off)
                    wp = jnp.where(do_flush, wp + SLAB, wp)

            @pl.when(b + 2 < NBP)
            def _():
                issue_idx(p, b + 2)
        return (off, wp)

    off, wp = lax.fori_loop(0, NBP // 2, scan_body,
                            (jnp.int32(0), jnp.int32(0)))

    total = wp + off
    nt = jnp.maximum((total + PADN - 1) // PADN, 1)
    for _g in range(PADN // LANES):
        sldst[pl.ds(off, LANES)] = jnp.full((LANES,), DUMMY, jnp.int32)
        ssrc[pl.ds(off, LANES)] = jnp.full((LANES,), N_NODES, jnp.int32)
        off = off + LANES

    do_flush = off >= SLAB

    @pl.when(do_flush)
    def _():
        flush(off, wp, 48)

    off = jnp.where(do_flush, off - SLAB, off)
    wp = jnp.where(do_flush, wp + SLAB, wp)

    for _g in range(CHUNK // LANES):
        sldst[pl.ds(off + _g * LANES, LANES)] = jnp.full((LANES,), DUMMY,
                                                         jnp.int32)
        ssrc[pl.ds(off + _g * LANES, LANES)] = jnp.full((LANES,), N_NODES,
                                                        jnp.int32)

    @pl.when(wp > 0)
    def _():
        for ch in range(NCH):
            drain_ones(ch)

    for ch in range(NCH):
        for g in range(CHUNK // LANES):
            lbuf[ch, pl.ds(g * LANES, LANES)] = (
                sldst[pl.ds(ch * CHUNK + g * LANES, LANES)])
    wpa = pl.multiple_of(wp, SLAB)
    pltpu.sync_copy(ssrc.at[pl.ds(0, SLAB)], psrc_hbm.at[w, pl.ds(wpa, SLAB)])
    pltpu.sync_copy(sldst.at[pl.ds(0, SLAB)], pldst_hbm.at[w, pl.ds(wpa, SLAB)])
    for ch in range(NCH):
        @pl.when(ch * CHUNK < off)
        def _():
            fire_ones(ch)
    for ch in range(NCH):
        @pl.when(ch * CHUNK < off)
        def _():
            drain_ones(ch)

    ncbuf[...] = jnp.full((16,), nt, jnp.int32)
    pltpu.sync_copy(ncbuf, pcnt_hbm.at[w])

    plsc.subcore_barrier()
    pltpu.sync_copy(acc_s.at[pl.ds(r0, WB)], cnt_hbm.at[pl.ds(base + r0, WB)])

    @pl.when(s == NS - 1)
    def _():
        pltpu.sync_copy(acc_s.at[pl.ds(NS * WB, 40)],
                        cnt_hbm.at[pl.ds(base + NS * WB, 40)])


@functools.partial(
    pl.kernel,
    out_type=jax.ShapeDtypeStruct((NPAD, DIM), jnp.float32),
    mesh=_sc_mesh,
    scratch_types=[
        pltpu.VMEM((2, NTCH * CHUNK), jnp.int32),
        pltpu.VMEM((2, NTCH, CHUNK), jnp.int32),
        pltpu.VMEM((3, CHUNK, DIM), jnp.float32),
        pltpu.VMEM((16,), jnp.int32),
        pltpu.SemaphoreType.DMA,
        pltpu.SemaphoreType.DMA,
        pltpu.SemaphoreType.DMA,
        pltpu.SemaphoreType.DMA,
        pltpu.SemaphoreType.DMA,
        pltpu.SemaphoreType.DMA,
        pltpu.SemaphoreType.DMA,
        pltpu.VMEM_SHARED((ACC_ROWS, DIM), jnp.float32),
    ],
    compiler_params=_sc_params,
)
def _scatter_kernel(y_hbm, psrc_hbm, pldst_hbm, pcnt_hbm, out_hbm,
                    src_v, ldst_v, rows_v, ncbuf, sem_i,
                    sem_g0, sem_g1, sem_g2, sem_s0, sem_s1, sem_s2, acc_s):
    sem_g = (sem_g0, sem_g1, sem_g2)
    sem_s = (sem_s0, sem_s1, sem_s2)
    c = lax.axis_index("c")
    s = lax.axis_index("s")
    w = c * NS + s
    base = c * HALF
    r0 = s * WB
    pltpu.sync_copy(y_hbm.at[pl.ds(base + r0, WB)], acc_s.at[pl.ds(r0, WB)])

    @pl.when(s == NS - 1)
    def _():
        pltpu.sync_copy(y_hbm.at[pl.ds(base + NS * WB, 40)],
                        acc_s.at[pl.ds(NS * WB, 40)])

    plsc.subcore_barrier()

    pltpu.sync_copy(pcnt_hbm.at[w], ncbuf)
    nt = jnp.max(ncbuf[...])

    def issue_idx(pb, t):
        pltpu.async_copy(psrc_hbm.at[w, pl.ds(t * NTCH * CHUNK, NTCH * CHUNK)],
                         src_v.at[pb], sem_i)
        pltpu.async_copy(pldst_hbm.at[w, pl.ds(t * NTCH, NTCH)],
                         ldst_v.at[pb], sem_i)

    def wait_idx(pb):
        pltpu.make_async_copy(psrc_hbm.at[w, pl.ds(0, NTCH * CHUNK)],
                              src_v.at[pb], sem_i).wait()
        pltpu.make_async_copy(pldst_hbm.at[w, pl.ds(0, NTCH)],
                              ldst_v.at[pb], sem_i).wait()

    def fire_gather(pb, r, u):
        pltpu.async_copy(y_hbm.at[src_v.at[pb, pl.ds(u * CHUNK, CHUNK)]],
                         rows_v.at[r], sem_g[r])

    def drain_gather(pb, r, u):
        pltpu.make_async_copy(y_hbm.at[src_v.at[pb, pl.ds(u * CHUNK, CHUNK)]],
                              rows_v.at[r], sem_g[r]).wait()

    def fire_scatter(pb, r, u):
        pltpu.async_copy(rows_v.at[r], acc_s.at[ldst_v.at[pb, u]], sem_s[r],
                         add=True)

    def drain_scatter(r):
        pltpu.make_async_copy(rows_v.at[r], acc_s.at[ldst_v.at[0, 0]],
                              sem_s[r]).wait()

    issue_idx(0, 0)

    def body(t, carry):
        pb = lax.rem(t, 2)
        wait_idx(pb)
        for u in range(NTCH):
            r = u % 3
            if u < 3:
                @pl.when(t > 0)
                def _():
                    drain_scatter(r)
            else:
                drain_scatter(r)
            fire_gather(pb, r, u)
            if u > 0:
                drain_gather(pb, (u - 1) % 3, u - 1)
                fire_scatter(pb, (u - 1) % 3, u - 1)
            else:
                @pl.when(t > 0)
                def _():
                    drain_gather(1 - pb, 2, NTCH - 1)
                    fire_scatter(1 - pb, 2, NTCH - 1)

        @pl.when(t + 1 < nt)
        def _():
            issue_idx(1 - pb, t + 1)
        return carry

    lax.fori_loop(0, nt, body, 0)
    pbl = lax.rem(nt - 1, 2)
    drain_gather(pbl, 2, NTCH - 1)
    fire_scatter(pbl, 2, NTCH - 1)
    for r in (0, 1, 2):
        drain_scatter(r)

    plsc.subcore_barrier()
    pltpu.sync_copy(acc_s.at[pl.ds(r0, WB)], out_hbm.at[pl.ds(base + r0, WB)])

    @pl.when(s == NS - 1)
    def _():
        pltpu.sync_copy(acc_s.at[pl.ds(NS * WB, 40)],
                        out_hbm.at[pl.ds(base + NS * WB, 40)])


def _tcA_body(x_ref, cnt_ref, w_ref, y_ref, dis_ref):
    dis = lax.rsqrt(cnt_ref[...][:, 0:1] + 1.0)
    xw = jnp.dot(x_ref[...], w_ref[...], preferred_element_type=jnp.float32)
    y_ref[...] = xw * dis
    dis_ref[...] = dis


def _tcB_body(acc_ref, dis_ref, b_ref, w_ref, out_ref, y_ref):
    dis = dis_ref[...]
    out = acc_ref[...] * dis + b_ref[0:1, :]
    out_ref[...] = out
    h = jnp.maximum(out, 0.0)
    y_ref[...] = jnp.dot(h, w_ref[...], preferred_element_type=jnp.float32) * dis


def _tcC_body(acc_ref, dis_ref, b_ref, out_ref):
    out_ref[...] = acc_ref[...] * dis_ref[...] + b_ref[0:1, :]


def _tcA(x_pad, cnt, W1):
    return pl.pallas_call(
        _tcA_body,
        grid=(GRID,),
        in_specs=[pl.BlockSpec((BLK, DIM), lambda i: (i, 0)),
                  pl.BlockSpec((BLK, LANES), lambda i: (i, 0)),
                  pl.BlockSpec((DIM, DIM), lambda i: (0, 0))],
        out_specs=[pl.BlockSpec((BLK, DIM), lambda i: (i, 0)),
                   pl.BlockSpec((BLK, 1), lambda i: (i, 0))],
        out_shape=[jax.ShapeDtypeStruct((NPAD, DIM), jnp.float32),
                   jax.ShapeDtypeStruct((NPAD, 1), jnp.float32)],
    )(x_pad, cnt, W1)


def _tcB(acc, dis, b1b, W2):
    return pl.pallas_call(
        _tcB_body,
        grid=(GRID,),
        in_specs=[pl.BlockSpec((BLK, DIM), lambda i: (i, 0)),
                  pl.BlockSpec((BLK, 1), lambda i: (i, 0)),
                  pl.BlockSpec((8, DIM), lambda i: (0, 0)),
                  pl.BlockSpec((DIM, DIM), lambda i: (0, 0))],
        out_specs=[pl.BlockSpec((BLK, DIM), lambda i: (i, 0)),
                   pl.BlockSpec((BLK, DIM), lambda i: (i, 0))],
        out_shape=[jax.ShapeDtypeStruct((NPAD, DIM), jnp.float32),
                   jax.ShapeDtypeStruct((NPAD, DIM), jnp.float32)],
    )(acc, dis, b1b, W2)


def _tcC(acc, dis, b2b):
    return pl.pallas_call(
        _tcC_body,
        grid=(GRID,),
        in_specs=[pl.BlockSpec((BLK, DIM), lambda i: (i, 0)),
                  pl.BlockSpec((BLK, 1), lambda i: (i, 0)),
                  pl.BlockSpec((8, DIM), lambda i: (0, 0))],
        out_specs=pl.BlockSpec((BLK, DIM), lambda i: (i, 0)),
        out_shape=jax.ShapeDtypeStruct((NPAD, DIM), jnp.float32),
    )(acc, dis, b2b)


def kernel(edge_index, edge_weight, emb_users, emb_items, W1, b1, W2, b2):
    del edge_weight
    src = edge_index[0].astype(jnp.int32)
    dst = edge_index[1].astype(jnp.int32)
    pad_e = E_PAD - E_EDGES
    src_p = jnp.concatenate([src, jnp.full((pad_e,), N_NODES, jnp.int32)])
    dst_p = jnp.concatenate([dst, jnp.full((pad_e,), 2 ** 28, jnp.int32)])

    x = jnp.concatenate([emb_users, emb_items], axis=0)
    x_pad = jnp.concatenate([x, jnp.zeros((NPAD - N_NODES, DIM), jnp.float32)])

    cnt, psrc, pldst, pcnt = _part_kernel(src_p, dst_p)
    pldst3 = pldst.reshape(NW, CAP_T // CHUNK, CHUNK)

    y1, dis = _tcA(x_pad, cnt, W1)
    acc1 = _scatter_kernel(y1, psrc, pldst3, pcnt)
    b1b = jnp.broadcast_to(b1.reshape(1, DIM), (8, DIM))
    out1, y2 = _tcB(acc1, dis, b1b, W2)
    acc2 = _scatter_kernel(y2, psrc, pldst3, pcnt)
    b2b = jnp.broadcast_to(b2.reshape(1, DIM), (8, DIM))
    out2 = _tcC(acc2, dis, b2b)

    return (x, out1[:N_NODES], out2[:N_NODES])
